# baseline (speedup 1.0000x reference)
import functools

import jax
import jax.numpy as jnp
from jax import lax
from jax.experimental import pallas as pl
from jax.experimental.pallas import tpu as pltpu
from jax.experimental.pallas import tpu_sc as plsc

B = 16384
D = 16
NC = 2
NS = 16
NW = NC * NS
BPW = B // NW
BM = 2048
NBLK = B // BM


def _sc_gather(i1, i2, emb3, emb):
    @functools.partial(
        pl.kernel,
        mesh=plsc.VectorSubcoreMesh(core_axis_name="c", subcore_axis_name="s"),
        out_type=[
            jax.ShapeDtypeStruct((B, D), jnp.float32),
            jax.ShapeDtypeStruct((B, D), jnp.float32),
        ],
        scratch_types=[
            pltpu.VMEM((BPW,), jnp.int32),
            pltpu.VMEM((BPW,), jnp.int32),
            pltpu.SemaphoreType.DMA,
        ],
    )
    def k(i1_hbm, i2_hbm, t1_hbm, t2_hbm, o1_hbm, o2_hbm,
          idx1_v, idx2_v, sem):
        wid = lax.axis_index("s") * NC + lax.axis_index("c")
        base = wid * BPW
        pltpu.sync_copy(i1_hbm.at[pl.ds(base, BPW)], idx1_v)
        pltpu.sync_copy(i2_hbm.at[pl.ds(base, BPW)], idx2_v)

        def body(c, carry):
            row0 = c * 16
            v1 = idx1_v[pl.ds(row0, 16)]
            v2 = idx2_v[pl.ds(row0, 16)]
            for l in range(16):
                pltpu.async_copy(
                    t1_hbm.at[v1[l]], o1_hbm.at[base + row0 + l], sem)
                pltpu.async_copy(
                    t2_hbm.at[v2[l]], o2_hbm.at[base + row0 + l], sem)
            return carry

        lax.fori_loop(0, BPW // 16, body, 0)
        pltpu.make_async_copy(
            t1_hbm.at[pl.ds(0, BPW)], o1_hbm.at[pl.ds(base, BPW)], sem).wait()
        pltpu.make_async_copy(
            t2_hbm.at[pl.ds(0, BPW)], o2_hbm.at[pl.ds(base, BPW)], sem).wait()

    return k(i1, i2, emb3, emb)


def _mlp(e1, e2, xo, W1a, W1b, W1c, b1, W2, b2, W3, b3):
    def body(e1_ref, e2_ref, xo_ref, w1a_ref, w1b_ref, w1c_ref, b1_ref,
             w2_ref, b2_ref, w3_ref, b3_ref, o_ref):
        h = (e1_ref[...] @ w1a_ref[...]
             + e2_ref[...] @ w1b_ref[...]
             + xo_ref[...] @ w1c_ref[...]
             + b1_ref[...])
        h = jnp.maximum(h, 0.0)
        h = jnp.maximum(h @ w2_ref[...] + b2_ref[...], 0.0)
        o_ref[...] = h @ w3_ref[...] + b3_ref[...]

    fixed = lambda *shape: pl.BlockSpec(shape, lambda i: (0,) * len(shape))
    return pl.pallas_call(
        body,
        grid=(NBLK,),
        in_specs=[
            pl.BlockSpec((BM, D), lambda i: (i, 0)),
            pl.BlockSpec((BM, D), lambda i: (i, 0)),
            pl.BlockSpec((BM, 64), lambda i: (i, 0)),
            fixed(D, 128),
            fixed(D, 128),
            fixed(64, 128),
            fixed(1, 128),
            fixed(128, 128),
            fixed(1, 128),
            fixed(128, 1),
            fixed(1, 1),
        ],
        out_specs=pl.BlockSpec((BM, 1), lambda i: (i, 0)),
        out_shape=jax.ShapeDtypeStruct((B, 1), jnp.float32),
    )(e1, e2, xo, W1a, W1b, W1c, b1, W2, b2, W3, b3)


def kernel(x, emb3, emb, W1, b1, W2, b2, W3, b3):
    i1 = x[:, 0].astype(jnp.int32)
    i2 = x[:, 1].astype(jnp.int32)
    xo = x[:, 2:]
    e1, e2 = _sc_gather(i1, i2, emb3, emb)
    return _mlp(e1, e2, xo,
                W1[:D], W1[D:2 * D], W1[2 * D:],
                b1.reshape(1, -1), W2, b2.reshape(1, -1),
                W3, b3.reshape(1, 1))


# speedup vs baseline: 1.9465x; 1.9465x over previous
import functools

import jax
import jax.numpy as jnp
from jax import lax
from jax.experimental import pallas as pl
from jax.experimental.pallas import tpu as pltpu
from jax.experimental.pallas import tpu_sc as plsc

B = 16384
D = 16
NC = 2
NS = 16
NW = NC * NS
BPW = B // NW
CH = 128
NCH = BPW // CH
BM = 2048
NBLK = B // BM


def _sc_gather_emb3(i1g, emb3):
    @functools.partial(
        pl.kernel,
        mesh=plsc.VectorSubcoreMesh(core_axis_name="c", subcore_axis_name="s"),
        compiler_params=pltpu.CompilerParams(use_tc_tiling_on_sc=False),
        out_type=jax.ShapeDtypeStruct((B, D), jnp.float32),
        scratch_types=[
            pltpu.VMEM((NCH, CH), jnp.int32),
            pltpu.VMEM((BPW, D), jnp.float32),
            pltpu.SemaphoreType.DMA,
        ],
    )
    def k(i1_hbm, t1_hbm, o1_hbm, idx_v, rows_v, sem):
        wid = lax.axis_index("s") * NC + lax.axis_index("c")
        base = wid * BPW
        pltpu.sync_copy(i1_hbm.at[wid], idx_v)
        copies = [
            pltpu.async_copy(
                t1_hbm.at[idx_v.at[j]], rows_v.at[pl.ds(j * CH, CH)], sem)
            for j in range(NCH)
        ]
        for c in copies:
            c.wait()
        pltpu.sync_copy(rows_v, o1_hbm.at[pl.ds(base, BPW)])

    return k(i1g, emb3)


def _tc_gather_mlp(i2, e1, xo, emb, W1a, W1b, W1c, b1, W2, b2, W3, b3):
    def body(i2_s, emb_hbm, e1_ref, xo_ref, w1a_ref, w1b_ref,
             w1c_ref, b1_ref, w2_ref, b2_ref, w3_ref, b3_ref, o_ref,
             e2b, sem):
        k = pl.program_id(0)
        base = k * BM

        def issue(r, carry):
            b = i2_s[base + r]
            pltpu.make_async_copy(
                emb_hbm.at[pl.ds(b, 1)], e2b.at[pl.ds(r, 1)], sem).start()
            return carry

        lax.fori_loop(0, BM, issue, 0, unroll=16)
        pltpu.make_async_copy(emb_hbm.at[pl.ds(0, BM)], e2b, sem).wait()

        h = (e1_ref[...] @ w1a_ref[...]
             + e2b[...] @ w1b_ref[...]
             + xo_ref[...] @ w1c_ref[...]
             + b1_ref[...])
        h = jnp.maximum(h, 0.0)
        h = jnp.maximum(h @ w2_ref[...] + b2_ref[...], 0.0)
        o_ref[...] = h @ w3_ref[...] + b3_ref[...]

    fixed = lambda *shape: pl.BlockSpec(shape, lambda i, *_: (0,) * len(shape))
    grid_spec = pltpu.PrefetchScalarGridSpec(
        num_scalar_prefetch=1,
        grid=(NBLK,),
        in_specs=[
            pl.BlockSpec(memory_space=pl.ANY),
            pl.BlockSpec((BM, D), lambda i, *_: (i, 0)),
            pl.BlockSpec((BM, 64), lambda i, *_: (i, 0)),
            fixed(D, 128),
            fixed(D, 128),
            fixed(64, 128),
            fixed(1, 128),
            fixed(128, 128),
            fixed(1, 128),
            fixed(128, 1),
            fixed(1, 1),
        ],
        out_specs=pl.BlockSpec((BM, 1), lambda i, *_: (i, 0)),
        scratch_shapes=[
            pltpu.VMEM((BM, D), jnp.float32),
            pltpu.SemaphoreType.DMA,
        ],
    )
    return pl.pallas_call(
        body,
        grid_spec=grid_spec,
        out_shape=jax.ShapeDtypeStruct((B, 1), jnp.float32),
    )(i2, emb, e1, xo, W1a, W1b, W1c, b1, W2, b2, W3, b3)


def kernel(x, emb3, emb, W1, b1, W2, b2, W3, b3):
    i1 = x[:, 0].astype(jnp.int32)
    i2 = x[:, 1].astype(jnp.int32)
    xo = x[:, 2:]
    e1 = _sc_gather_emb3(i1.reshape(NW, NCH, CH), emb3)
    return _tc_gather_mlp(i2, e1, xo, emb,
                          W1[:D], W1[D:2 * D], W1[2 * D:],
                          b1.reshape(1, -1), W2, b2.reshape(1, -1),
                          W3, b3.reshape(1, 1))
